# baseline (speedup 1.0000x reference)
import jax
import jax.numpy as jnp
from jax import lax
from jax.experimental import pallas as pl
from jax.experimental.pallas import tpu as pltpu
from jax.experimental.pallas import tpu_sc as plsc

N = 10000
E = 320000
D_IN = 128
D_OUT = 64

NN = 2 * N
NP = 20480
DUMMY = NN

NC = 2
NS = 16
NW = NC * NS
B = 128
KB = 1
BOP = KB * B
CH = 162
NBUF = 3
PF = 2
PW = CH * BOP
EP = NW * PW

ROWS_PER_TILE = NP // NS


def _zero_bf16_rows(ref, nrows, ncols):
    z = jnp.zeros((32,), jnp.bfloat16)

    def body(i, carry):
        for j in range(ncols // 32):
            ref[i, pl.ds(j * 32, 32)] = z
        return carry

    lax.fori_loop(0, nrows, body, 0)


def _make_sc_scatter(with_deg: bool):
    mesh = plsc.VectorSubcoreMesh(core_axis_name="c", subcore_axis_name="s")
    out_type = [jax.ShapeDtypeStruct((NC, NP, D_OUT), jnp.bfloat16)]
    if with_deg:
        out_type.append(jax.ShapeDtypeStruct((NC, NP), jnp.float32))
    scratch = [
        pltpu.VMEM((CH, BOP), jnp.int32),
        pltpu.VMEM((CH, BOP), jnp.int32),
        [pltpu.VMEM((BOP, D_OUT), jnp.bfloat16) for _ in range(NBUF)],
        pltpu.VMEM_SHARED((NP, D_OUT), jnp.bfloat16),
        [pltpu.SemaphoreType.DMA for _ in range(NBUF)],
    ]
    if with_deg:
        scratch.append(pltpu.VMEM((BOP,), jnp.float32))
        scratch.append(pltpu.VMEM_SHARED((NP,), jnp.float32))

    def body(xn_hbm, row_hbm, col_hbm, s_out, *rest):
        if with_deg:
            (deg_out, row_v, col_v, bufs, s_sh, gsems,
             ones_v, deg_sh) = rest
        else:
            (row_v, col_v, bufs, s_sh, gsems) = rest
        buf0 = bufs[0]
        cid = lax.axis_index("c")
        sid = lax.axis_index("s")
        wid = sid * NC + cid

        pltpu.sync_copy(row_hbm.at[wid], row_v)
        pltpu.sync_copy(col_hbm.at[wid], col_v)

        ZR = 256
        _zero_bf16_rows(buf0, ZR, D_OUT)
        for i in range(ROWS_PER_TILE // ZR):
            pltpu.sync_copy(
                buf0.at[pl.ds(0, ZR)],
                s_sh.at[pl.ds(sid * ROWS_PER_TILE + i * ZR, ZR)])
        if with_deg:
            zf = jnp.zeros((16,), jnp.float32)
            for k in range(BOP // 16):
                ones_v[pl.ds(k * 16, 16)] = zf
            pltpu.sync_copy(ones_v,
                            deg_sh.at[pl.ds(sid * ROWS_PER_TILE, BOP)])
            pltpu.sync_copy(
                ones_v.at[pl.ds(0, ROWS_PER_TILE - BOP)],
                deg_sh.at[pl.ds(sid * ROWS_PER_TILE + BOP,
                                ROWS_PER_TILE - BOP)])
            one = jnp.ones((16,), jnp.float32)
            for k in range(BOP // 16):
                ones_v[pl.ds(k * 16, 16)] = one
        plsc.subcore_barrier()

        for b in range(PF):
            pltpu.async_copy(xn_hbm.at[row_v.at[b]], bufs[b], gsems[b])

        def chunk(g, carry):
            for b in range(NBUF):
                j = g * NBUF + b
                cur, gsem = bufs[b], gsems[b]
                nb = (b + PF) % NBUF
                pltpu.make_async_copy(xn_hbm.at[row_v.at[j]], cur, gsem).wait()

                @pl.when(j + PF < CH)
                def _():
                    pltpu.async_copy(
                        xn_hbm.at[row_v.at[j + PF]], bufs[nb], gsems[nb])

                pltpu.sync_copy(cur, s_sh.at[col_v.at[j]], add=True)
                if with_deg:
                    pltpu.sync_copy(ones_v, deg_sh.at[col_v.at[j]], add=True)
            return carry

        lax.fori_loop(0, CH // NBUF, chunk, 0)
        plsc.subcore_barrier()

        r0 = sid * ROWS_PER_TILE
        pltpu.sync_copy(s_sh.at[pl.ds(r0, ROWS_PER_TILE)],
                        s_out.at[cid, pl.ds(r0, ROWS_PER_TILE)])
        if with_deg:
            pltpu.sync_copy(deg_sh.at[pl.ds(r0, ROWS_PER_TILE)],
                            deg_out.at[cid, pl.ds(r0, ROWS_PER_TILE)])

    return pl.kernel(body, out_type=out_type, mesh=mesh,
                     scratch_types=scratch,
                     compiler_params=pltpu.CompilerParams(
                         use_tc_tiling_on_sc=False))


BM = 1024


def _mm_body(x_ref, w_ref, o_ref):
    o_ref[...] = lax.dot_general(
        x_ref[...], w_ref[...], (((1,), (1,)), ((), ())),
        preferred_element_type=jnp.float32).astype(jnp.bfloat16)


def _tc_xn(x, w):
    d = x.shape[1]
    return pl.pallas_call(
        _mm_body,
        grid=(NP // BM,),
        in_specs=[
            pl.BlockSpec((BM, d), lambda i: (i, 0)),
            pl.BlockSpec((D_OUT, d), lambda i: (0, 0)),
        ],
        out_specs=pl.BlockSpec((BM, D_OUT), lambda i: (i, 0)),
        out_shape=jax.ShapeDtypeStruct((NP, D_OUT), jnp.bfloat16),
    )(x, w)


def _make_tc_combine(emit_next_xn: bool):
    def body(x_ref, ws_ref, s_ref, deg_ref, *rest):
        if emit_next_xn:
            we_ref, h_ref, xn_ref = rest
        else:
            (h_ref,) = rest
        xs = lax.dot_general(
            x_ref[...], ws_ref[...], (((1,), (1,)), ((), ())),
            preferred_element_type=jnp.float32)
        s = s_ref[0].astype(jnp.float32) + s_ref[1].astype(jnp.float32)
        deg = deg_ref[0] + deg_ref[1]
        inv = jnp.where(deg > 0, 1.0 / deg, 0.0)
        aggr = s * inv[:, None]
        h = jax.nn.sigmoid(jnp.concatenate([xs, aggr], axis=1))
        h_ref[...] = h
        if emit_next_xn:
            xn_ref[...] = lax.dot_general(
                h, we_ref[...], (((1,), (1,)), ((), ())),
                preferred_element_type=jnp.float32).astype(jnp.bfloat16)

    def run(x, w_self, s, deg, w_edge_next=None):
        d = x.shape[1]
        in_specs = [
            pl.BlockSpec((BM, d), lambda i: (i, 0)),
            pl.BlockSpec((D_OUT, d), lambda i: (0, 0)),
            pl.BlockSpec((NC, BM, D_OUT), lambda i: (0, i, 0)),
            pl.BlockSpec((NC, BM), lambda i: (0, i)),
        ]
        args = [x, w_self, s, deg]
        out_specs = [pl.BlockSpec((BM, 2 * D_OUT), lambda i: (i, 0))]
        out_shape = [jax.ShapeDtypeStruct((NP, 2 * D_OUT), jnp.float32)]
        if emit_next_xn:
            in_specs.append(pl.BlockSpec((D_OUT, 2 * D_OUT), lambda i: (0, 0)))
            args.append(w_edge_next)
            out_specs.append(pl.BlockSpec((BM, D_OUT), lambda i: (i, 0)))
            out_shape.append(jax.ShapeDtypeStruct((NP, D_OUT), jnp.bfloat16))
        return pl.pallas_call(
            body,
            grid=(NP // BM,),
            in_specs=in_specs,
            out_specs=out_specs,
            out_shape=out_shape,
        )(*args)

    return run


_sc_scatter_deg = _make_sc_scatter(with_deg=True)
_sc_scatter = _make_sc_scatter(with_deg=False)
_tc_combine_xn = _make_tc_combine(emit_next_xn=True)
_tc_combine = _make_tc_combine(emit_next_xn=False)


def kernel(x1, edge_index1, x2, edge_index2,
           W_edge0, W_self0, W_edge1, W_self1):
    row = jnp.concatenate([edge_index1[0], edge_index2[0] + N])
    col = jnp.concatenate([edge_index1[1], edge_index2[1] + N])
    pad = EP - 2 * E
    row = jnp.concatenate([row, jnp.zeros((pad,), jnp.int32)])
    col = jnp.concatenate([col, jnp.full((pad,), DUMMY, jnp.int32)])
    row = row.reshape(NW, CH, BOP)
    col = col.reshape(NW, CH, BOP)
    x = jnp.zeros((NP, D_IN), jnp.float32).at[:N].set(x1).at[N:NN].set(x2)

    xn0 = _tc_xn(x, W_edge0)
    s0, deg = _sc_scatter_deg(xn0, row, col)
    h, xn1 = _tc_combine_xn(x, W_self0, s0, deg, W_edge1)

    (s1,) = _sc_scatter(xn1, row, col)
    (out,) = _tc_combine(h, W_self1, s1, deg)

    return out[:N], out[N:NN]


# speedup vs baseline: 1.3384x; 1.3384x over previous
import jax
import jax.numpy as jnp
from jax import lax
from jax.experimental import pallas as pl
from jax.experimental.pallas import tpu as pltpu
from jax.experimental.pallas import tpu_sc as plsc

N = 10000
E = 320000
D_IN = 128
D_OUT = 64

NN = 2 * N
NP = 20480
DUMMY = NN

NC = 2
NS = 16
NW = NC * NS
B = 128
BOP = B
CH0 = 188
CH1 = 128
CHMAX = max(CH0, CH1)
EP = NS * (CH0 + CH1) * BOP

ROWS_PER_TILE = NP // NS


def _zero_bf16_rows(ref, nrows, ncols):
    z = jnp.zeros((32,), jnp.bfloat16)

    def body(i, carry):
        for j in range(ncols // 32):
            ref[i, pl.ds(j * 32, 32)] = z
        return carry

    lax.fori_loop(0, nrows, body, 0)


def _make_sc_scatter(with_deg: bool):
    mesh = plsc.VectorSubcoreMesh(core_axis_name="c", subcore_axis_name="s")
    out_type = [jax.ShapeDtypeStruct((NC, NP, D_OUT), jnp.bfloat16)]
    if with_deg:
        out_type.append(jax.ShapeDtypeStruct((NC, NP), jnp.float32))
    scratch = [
        pltpu.VMEM((CHMAX, BOP), jnp.int32),
        pltpu.VMEM((CHMAX, BOP), jnp.int32),
        [pltpu.VMEM((BOP, D_OUT), jnp.bfloat16) for _ in range(2)],
        pltpu.VMEM_SHARED((NP, D_OUT), jnp.bfloat16),
        [pltpu.SemaphoreType.DMA for _ in range(2)],
    ]
    if with_deg:
        scratch.append(pltpu.VMEM((BOP,), jnp.float32))
        scratch.append(pltpu.VMEM_SHARED((NP,), jnp.float32))

    def body(xn_hbm, row_hbm, col_hbm, s_out, *rest):
        if with_deg:
            (deg_out, row_v, col_v, bufs, s_sh, gsems,
             ones_v, deg_sh) = rest
        else:
            (row_v, col_v, bufs, s_sh, gsems) = rest
        buf0 = bufs[0]
        cid = lax.axis_index("c")
        sid = lax.axis_index("s")
        nch = jnp.where(cid == 0, CH0, CH1)

        pltpu.sync_copy(row_hbm.at[cid, sid], row_v)
        pltpu.sync_copy(col_hbm.at[cid, sid], col_v)

        ZR = 256
        _zero_bf16_rows(buf0, ZR, D_OUT)
        for i in range(ROWS_PER_TILE // ZR):
            pltpu.sync_copy(
                buf0.at[pl.ds(0, ZR)],
                s_sh.at[pl.ds(sid * ROWS_PER_TILE + i * ZR, ZR)])
        if with_deg:
            zf = jnp.zeros((16,), jnp.float32)
            for k in range(BOP // 16):
                ones_v[pl.ds(k * 16, 16)] = zf
            pltpu.sync_copy(ones_v,
                            deg_sh.at[pl.ds(sid * ROWS_PER_TILE, BOP)])
            pltpu.sync_copy(
                ones_v.at[pl.ds(0, ROWS_PER_TILE - BOP)],
                deg_sh.at[pl.ds(sid * ROWS_PER_TILE + BOP,
                                ROWS_PER_TILE - BOP)])
            one = jnp.ones((16,), jnp.float32)
            for k in range(BOP // 16):
                ones_v[pl.ds(k * 16, 16)] = one
        plsc.subcore_barrier()

        pltpu.async_copy(xn_hbm.at[row_v.at[0]], bufs[0], gsems[0])

        def chunk(g, carry):
            for b in range(2):
                j = g * 2 + b
                cur, gsem = bufs[b], gsems[b]
                nxt, ngsem = bufs[1 - b], gsems[1 - b]
                pltpu.make_async_copy(xn_hbm.at[row_v.at[j]], cur, gsem).wait()

                @pl.when(j + 1 < nch)
                def _():
                    pltpu.async_copy(xn_hbm.at[row_v.at[j + 1]], nxt, ngsem)

                pltpu.sync_copy(cur, s_sh.at[col_v.at[j]], add=True)
                if with_deg:
                    pltpu.sync_copy(ones_v, deg_sh.at[col_v.at[j]], add=True)
            return carry

        lax.fori_loop(0, nch // 2, chunk, 0)
        plsc.subcore_barrier()

        r0 = sid * ROWS_PER_TILE
        pltpu.sync_copy(s_sh.at[pl.ds(r0, ROWS_PER_TILE)],
                        s_out.at[cid, pl.ds(r0, ROWS_PER_TILE)])
        if with_deg:
            pltpu.sync_copy(deg_sh.at[pl.ds(r0, ROWS_PER_TILE)],
                            deg_out.at[cid, pl.ds(r0, ROWS_PER_TILE)])

    return pl.kernel(body, out_type=out_type, mesh=mesh,
                     scratch_types=scratch,
                     compiler_params=pltpu.CompilerParams(
                         use_tc_tiling_on_sc=False))


BM = 1024


def _mm_body(x_ref, w_ref, o_ref):
    o_ref[...] = lax.dot_general(
        x_ref[...], w_ref[...], (((1,), (1,)), ((), ())),
        preferred_element_type=jnp.float32).astype(jnp.bfloat16)


def _tc_xn(x, w):
    d = x.shape[1]
    return pl.pallas_call(
        _mm_body,
        grid=(NP // BM,),
        in_specs=[
            pl.BlockSpec((BM, d), lambda i: (i, 0)),
            pl.BlockSpec((D_OUT, d), lambda i: (0, 0)),
        ],
        out_specs=pl.BlockSpec((BM, D_OUT), lambda i: (i, 0)),
        out_shape=jax.ShapeDtypeStruct((NP, D_OUT), jnp.bfloat16),
    )(x, w)


def _make_tc_combine(emit_next_xn: bool):
    def body(x_ref, ws_ref, s_ref, deg_ref, *rest):
        if emit_next_xn:
            we_ref, h_ref, xn_ref = rest
        else:
            (h_ref,) = rest
        xs = lax.dot_general(
            x_ref[...], ws_ref[...], (((1,), (1,)), ((), ())),
            preferred_element_type=jnp.float32)
        s = s_ref[0].astype(jnp.float32) + s_ref[1].astype(jnp.float32)
        deg = deg_ref[0] + deg_ref[1]
        inv = jnp.where(deg > 0, 1.0 / deg, 0.0)
        aggr = s * inv[:, None]
        h = jax.nn.sigmoid(jnp.concatenate([xs, aggr], axis=1))
        h_ref[...] = h
        if emit_next_xn:
            xn_ref[...] = lax.dot_general(
                h, we_ref[...], (((1,), (1,)), ((), ())),
                preferred_element_type=jnp.float32).astype(jnp.bfloat16)

    def run(x, w_self, s, deg, w_edge_next=None):
        d = x.shape[1]
        in_specs = [
            pl.BlockSpec((BM, d), lambda i: (i, 0)),
            pl.BlockSpec((D_OUT, d), lambda i: (0, 0)),
            pl.BlockSpec((NC, BM, D_OUT), lambda i: (0, i, 0)),
            pl.BlockSpec((NC, BM), lambda i: (0, i)),
        ]
        args = [x, w_self, s, deg]
        out_specs = [pl.BlockSpec((BM, 2 * D_OUT), lambda i: (i, 0))]
        out_shape = [jax.ShapeDtypeStruct((NP, 2 * D_OUT), jnp.float32)]
        if emit_next_xn:
            in_specs.append(pl.BlockSpec((D_OUT, 2 * D_OUT), lambda i: (0, 0)))
            args.append(w_edge_next)
            out_specs.append(pl.BlockSpec((BM, D_OUT), lambda i: (i, 0)))
            out_shape.append(jax.ShapeDtypeStruct((NP, D_OUT), jnp.bfloat16))
        return pl.pallas_call(
            body,
            grid=(NP // BM,),
            in_specs=in_specs,
            out_specs=out_specs,
            out_shape=out_shape,
        )(*args)

    return run


_sc_scatter_deg = _make_sc_scatter(with_deg=True)
_sc_scatter = _make_sc_scatter(with_deg=False)
_tc_combine_xn = _make_tc_combine(emit_next_xn=True)
_tc_combine = _make_tc_combine(emit_next_xn=False)


def kernel(x1, edge_index1, x2, edge_index2,
           W_edge0, W_self0, W_edge1, W_self1):
    row = jnp.concatenate([edge_index1[0], edge_index2[0] + N])
    col = jnp.concatenate([edge_index1[1], edge_index2[1] + N])
    pad = EP - 2 * E
    row = jnp.concatenate([row, jnp.zeros((pad,), jnp.int32)])
    col = jnp.concatenate([col, jnp.full((pad,), DUMMY, jnp.int32)])

    def _layout(a):
        c0 = a[:NS * CH0 * BOP].reshape(NS, CH0, BOP)
        c1 = a[NS * CH0 * BOP:].reshape(NS, CH1, BOP)
        c0 = jnp.pad(c0, ((0, 0), (0, CHMAX - CH0), (0, 0)))
        c1 = jnp.pad(c1, ((0, 0), (0, CHMAX - CH1), (0, 0)))
        return jnp.stack([c0, c1])

    row = _layout(row)
    col = _layout(col)
    x = jnp.zeros((NP, D_IN), jnp.float32).at[:N].set(x1).at[N:NN].set(x2)

    xn0 = _tc_xn(x, W_edge0)
    s0, deg = _sc_scatter_deg(xn0, row, col)
    h, xn1 = _tc_combine_xn(x, W_self0, s0, deg, W_edge1)

    (s1,) = _sc_scatter(xn1, row, col)
    (out,) = _tc_combine(h, W_self1, s1, deg)

    return out[:N], out[N:NN]


# speedup vs baseline: 1.3471x; 1.0065x over previous
import jax
import jax.numpy as jnp
from jax import lax
from jax.experimental import pallas as pl
from jax.experimental.pallas import tpu as pltpu
from jax.experimental.pallas import tpu_sc as plsc

N = 10000
E = 320000
D_IN = 128
D_OUT = 64

NN = 2 * N
NP = 20480
DUMMY = NN

NC = 2
NS = 16
NW = NC * NS
B = 128
BOP = B
CH = 158
PW = CH * BOP
EP = NW * PW

ROWS_PER_TILE = NP // NS


def _zero_bf16_rows(ref, nrows, ncols):
    z = jnp.zeros((32,), jnp.bfloat16)

    def body(i, carry):
        for j in range(ncols // 32):
            ref[i, pl.ds(j * 32, 32)] = z
        return carry

    lax.fori_loop(0, nrows, body, 0)


def _make_sc_scatter(with_deg: bool):
    mesh = plsc.VectorSubcoreMesh(core_axis_name="c", subcore_axis_name="s")
    out_type = [jax.ShapeDtypeStruct((NC, NP, D_OUT), jnp.bfloat16)]
    if with_deg:
        out_type.append(jax.ShapeDtypeStruct((NC, NP), jnp.float32))
    scratch = [
        pltpu.VMEM((CH, BOP), jnp.int32),
        pltpu.VMEM((CH, BOP), jnp.int32),
        [pltpu.VMEM((BOP, D_OUT), jnp.bfloat16) for _ in range(2)],
        pltpu.VMEM_SHARED((NP, D_OUT), jnp.bfloat16),
        [pltpu.SemaphoreType.DMA for _ in range(2)],
    ]
    if with_deg:
        scratch.append(pltpu.VMEM((BOP,), jnp.float32))
        scratch.append(pltpu.VMEM_SHARED((NP,), jnp.float32))

    def body(xn_hbm, row_hbm, col_hbm, s_out, *rest):
        if with_deg:
            (deg_out, row_v, col_v, bufs, s_sh, gsems,
             ones_v, deg_sh) = rest
        else:
            (row_v, col_v, bufs, s_sh, gsems) = rest
        buf0 = bufs[0]
        cid = lax.axis_index("c")
        sid = lax.axis_index("s")
        wid = sid * NC + cid

        pltpu.sync_copy(row_hbm.at[wid], row_v)
        pltpu.sync_copy(col_hbm.at[wid], col_v)

        _zero_bf16_rows(buf0, BOP, D_OUT)
        for i in range(ROWS_PER_TILE // BOP):
            pltpu.sync_copy(
                buf0, s_sh.at[pl.ds(sid * ROWS_PER_TILE + i * BOP, BOP)])
        if with_deg:
            zf = jnp.zeros((16,), jnp.float32)
            for k in range(BOP // 16):
                ones_v[pl.ds(k * 16, 16)] = zf
            for i in range(ROWS_PER_TILE // BOP):
                pltpu.sync_copy(
                    ones_v,
                    deg_sh.at[pl.ds(sid * ROWS_PER_TILE + i * BOP, BOP)])
            one = jnp.ones((16,), jnp.float32)
            for k in range(BOP // 16):
                ones_v[pl.ds(k * 16, 16)] = one
        plsc.subcore_barrier()

        pltpu.async_copy(xn_hbm.at[row_v.at[0]], bufs[0], gsems[0])

        def chunk(g, carry):
            for b in range(2):
                j = g * 2 + b
                cur, gsem = bufs[b], gsems[b]
                nxt, ngsem = bufs[1 - b], gsems[1 - b]
                pltpu.make_async_copy(xn_hbm.at[row_v.at[j]], cur, gsem).wait()

                @pl.when(j + 1 < CH)
                def _():
                    pltpu.async_copy(xn_hbm.at[row_v.at[j + 1]], nxt, ngsem)

                pltpu.sync_copy(cur, s_sh.at[col_v.at[j]], add=True)
                if with_deg:
                    pltpu.sync_copy(ones_v, deg_sh.at[col_v.at[j]], add=True)
            return carry

        lax.fori_loop(0, CH // 2, chunk, 0)
        plsc.subcore_barrier()

        r0 = sid * ROWS_PER_TILE
        pltpu.sync_copy(s_sh.at[pl.ds(r0, ROWS_PER_TILE)],
                        s_out.at[cid, pl.ds(r0, ROWS_PER_TILE)])
        if with_deg:
            pltpu.sync_copy(deg_sh.at[pl.ds(r0, ROWS_PER_TILE)],
                            deg_out.at[cid, pl.ds(r0, ROWS_PER_TILE)])

    return pl.kernel(body, out_type=out_type, mesh=mesh,
                     scratch_types=scratch,
                     compiler_params=pltpu.CompilerParams(
                         use_tc_tiling_on_sc=False))


BM = 1024


def _mm_body(x_ref, w_ref, o_ref):
    o_ref[...] = lax.dot_general(
        x_ref[...], w_ref[...], (((1,), (1,)), ((), ())),
        preferred_element_type=jnp.float32).astype(jnp.bfloat16)


def _tc_xn(x, w):
    d = x.shape[1]
    return pl.pallas_call(
        _mm_body,
        grid=(NP // BM,),
        in_specs=[
            pl.BlockSpec((BM, d), lambda i: (i, 0)),
            pl.BlockSpec((D_OUT, d), lambda i: (0, 0)),
        ],
        out_specs=pl.BlockSpec((BM, D_OUT), lambda i: (i, 0)),
        out_shape=jax.ShapeDtypeStruct((NP, D_OUT), jnp.bfloat16),
    )(x, w)


def _make_tc_combine(emit_next_xn: bool):
    def body(x_ref, ws_ref, s_ref, deg_ref, *rest):
        if emit_next_xn:
            we_ref, h_ref, xn_ref = rest
        else:
            (h_ref,) = rest
        xs = lax.dot_general(
            x_ref[...], ws_ref[...], (((1,), (1,)), ((), ())),
            preferred_element_type=jnp.float32)
        s = s_ref[0].astype(jnp.float32) + s_ref[1].astype(jnp.float32)
        deg = deg_ref[0] + deg_ref[1]
        inv = jnp.where(deg > 0, 1.0 / deg, 0.0)
        aggr = s * inv[:, None]
        h = jax.nn.sigmoid(jnp.concatenate([xs, aggr], axis=1))
        h_ref[...] = h
        if emit_next_xn:
            xn_ref[...] = lax.dot_general(
                h, we_ref[...], (((1,), (1,)), ((), ())),
                preferred_element_type=jnp.float32).astype(jnp.bfloat16)

    def run(x, w_self, s, deg, w_edge_next=None):
        d = x.shape[1]
        in_specs = [
            pl.BlockSpec((BM, d), lambda i: (i, 0)),
            pl.BlockSpec((D_OUT, d), lambda i: (0, 0)),
            pl.BlockSpec((NC, BM, D_OUT), lambda i: (0, i, 0)),
            pl.BlockSpec((NC, BM), lambda i: (0, i)),
        ]
        args = [x, w_self, s, deg]
        out_specs = [pl.BlockSpec((BM, 2 * D_OUT), lambda i: (i, 0))]
        out_shape = [jax.ShapeDtypeStruct((NP, 2 * D_OUT), jnp.float32)]
        if emit_next_xn:
            in_specs.append(pl.BlockSpec((D_OUT, 2 * D_OUT), lambda i: (0, 0)))
            args.append(w_edge_next)
            out_specs.append(pl.BlockSpec((BM, D_OUT), lambda i: (i, 0)))
            out_shape.append(jax.ShapeDtypeStruct((NP, D_OUT), jnp.bfloat16))
        return pl.pallas_call(
            body,
            grid=(NP // BM,),
            in_specs=in_specs,
            out_specs=out_specs,
            out_shape=out_shape,
        )(*args)

    return run


_sc_scatter_deg = _make_sc_scatter(with_deg=True)
_sc_scatter = _make_sc_scatter(with_deg=False)
_tc_combine_xn = _make_tc_combine(emit_next_xn=True)
_tc_combine = _make_tc_combine(emit_next_xn=False)


def kernel(x1, edge_index1, x2, edge_index2,
           W_edge0, W_self0, W_edge1, W_self1):
    row = jnp.concatenate([edge_index1[0], edge_index2[0] + N])
    col = jnp.concatenate([edge_index1[1], edge_index2[1] + N])
    pad = EP - 2 * E
    row = jnp.concatenate([row, jnp.zeros((pad,), jnp.int32)])
    col = jnp.concatenate([col, jnp.full((pad,), DUMMY, jnp.int32)])
    row = row.reshape(NW, CH, BOP)
    col = col.reshape(NW, CH, BOP)
    x = jnp.zeros((NP, D_IN), jnp.float32).at[:N].set(x1).at[N:NN].set(x2)

    xn0 = _tc_xn(x, W_edge0)
    s0, deg = _sc_scatter_deg(xn0, row, col)
    h, xn1 = _tc_combine_xn(x, W_self0, s0, deg, W_edge1)

    (s1,) = _sc_scatter(xn1, row, col)
    (out,) = _tc_combine(h, W_self1, s1, deg)

    return out[:N], out[N:NN]
